# baseline (speedup 1.0000x reference)
import functools

import jax
import jax.numpy as jnp
from jax import lax
from jax.experimental import pallas as pl
from jax.experimental.pallas import tpu as pltpu
from jax.experimental.pallas import tpu_sc as plsc

_N = 1048576
_NW = 32
_RW = _N // _NW
_R = 4096
_NCHUNK = _RW // _R
_GROUPS = _R // 16


def _sc_partials(a_flat, b_flat, lab_flat):
    mesh = plsc.VectorSubcoreMesh(core_axis_name="c", subcore_axis_name="s")

    @functools.partial(
        pl.kernel,
        mesh=mesh,
        out_type=[
            jax.ShapeDtypeStruct((_NW * 16,), jnp.float32),
            jax.ShapeDtypeStruct((_NW * 16,), jnp.float32),
        ],
        scratch_types=[
            pltpu.VMEM((4 * _R,), jnp.float32),
            pltpu.VMEM((4 * _R,), jnp.float32),
            pltpu.VMEM((_R,), jnp.float32),
            pltpu.VMEM((16,), jnp.float32),
            pltpu.VMEM((16,), jnp.float32),
        ],
    )
    def k(a_hbm, b_hbm, lab_hbm, acc_out, cnt_out, a_v, b_v, l_v, acc_v, cnt_v):
        wid = lax.axis_index("s") * 2 + lax.axis_index("c")
        rbase = wid * _RW
        lane = lax.iota(jnp.int32, 16)
        sub = lane >> 2

        def chunk_body(ci, carry):
            acc, cnt = carry
            row0 = rbase + ci * _R
            pltpu.sync_copy(a_hbm.at[pl.ds(row0 * 4, 4 * _R)], a_v)
            pltpu.sync_copy(b_hbm.at[pl.ds(row0 * 4, 4 * _R)], b_v)
            pltpu.sync_copy(lab_hbm.at[pl.ds(row0, _R)], l_v)

            def g_body(g, carry2):
                acc, cnt = carry2
                lab = l_v[pl.ds(g * 16, 16)]
                m = lab * lab
                cnt = cnt + m
                for c in range(4):
                    av = a_v[pl.ds(g * 64 + c * 16, 16)]
                    bv = b_v[pl.ds(g * 64 + c * 16, 16)]
                    d = av - bv
                    mm = m.at[sub + 4 * c].get(mode="promise_in_bounds")
                    acc = acc + d * d * mm
                return acc, cnt

            return lax.fori_loop(0, _GROUPS, g_body, (acc, cnt))

        acc0 = jnp.zeros((16,), jnp.float32)
        cnt0 = jnp.zeros((16,), jnp.float32)
        acc, cnt = lax.fori_loop(0, _NCHUNK, chunk_body, (acc0, cnt0))
        acc_v[...] = acc
        cnt_v[...] = cnt
        pltpu.sync_copy(acc_v, acc_out.at[pl.ds(wid * 16, 16)])
        pltpu.sync_copy(cnt_v, cnt_out.at[pl.ds(wid * 16, 16)])

    return k(a_flat, b_flat, lab_flat)


def kernel(bbox_out, bbox_target, label):
    a = bbox_out.reshape(-1)
    b = bbox_target.reshape(-1)
    lab = label.reshape(-1)
    acc, cnt = _sc_partials(a, b, lab)
    total = jnp.sum(acc)
    keep_num = jnp.sum(cnt)
    return total / keep_num
